# baseline (speedup 1.0000x reference)
import jax
import jax.numpy as jnp
from jax import lax
from jax.experimental import pallas as pl
from jax.experimental.pallas import tpu as pltpu
from jax.experimental.pallas import tpu_sc as plsc

_B = 16384
_K = 20
_D = 64
_NC = 2
_NS = 16
_NW = _NC * _NS
_BPW = _B // _NW
_T = 32
_NT = _BPW // _T
_TK = _T * _K
_NCHUNK = _TK // 128
_OC = 32


def _sc_logits_kernel(src_hbm, pos_hbm, negs_hbm, emb_hbm, ctx_hbm, out_hbm,
                      sidx, pidx, nidx,
                      erows0, erows1, prows0, prows1, nrows0, nrows1,
                      outv, scr, sem0, sem1):
    wid = lax.axis_index("s") * _NC + lax.axis_index("c")
    base = wid * _BPW
    col0 = lax.iota(jnp.int32, 16) * 16
    bufs = ((erows0, prows0, nrows0, sem0),
            (erows1, prows1, nrows1, sem1))

    pltpu.sync_copy(src_hbm.at[pl.ds(base, _BPW)], sidx)
    pltpu.sync_copy(pos_hbm.at[pl.ds(base, _BPW)], pidx)
    pltpu.sync_copy(negs_hbm.at[pl.ds(base * _K, _BPW * _K)], nidx)

    def gather_descs(t, b):
        erows, prows, nrows, sem = bufs[b]
        descs = [(emb_hbm.at[sidx.at[pl.ds(t * _T, _T)]], erows, sem),
                 (ctx_hbm.at[pidx.at[pl.ds(t * _T, _T)]], prows, sem)]
        for c in range(_NCHUNK):
            descs.append(
                (ctx_hbm.at[nidx.at[pl.ds(t * _TK + c * 128, 128)]],
                 nrows.at[pl.ds(c * 128, 128)], sem))
        return descs

    def issue(t, b):
        for d in gather_descs(t, b):
            pltpu.async_copy(*d)

    def drain(t, b):
        for d in gather_descs(t, b):
            pltpu.make_async_copy(*d).wait()

    def compute(t, b):
        erows, prows, nrows, _ = bufs[b]
        tb = base + t * _T

        def sample_body(i, carry2):
            fmt = plsc.PackFormat.INTERLEAVED
            ea = plsc.unpack(erows[i, pl.ds(0, 32)], format=fmt)
            eb = plsc.unpack(erows[i, pl.ds(32, 32)], format=fmt)
            e = [ea[0], ea[1], eb[0], eb[1]]

            def dot_to(j, load_r):
                ra = plsc.unpack(load_r(0), format=fmt)
                rb = plsc.unpack(load_r(1), format=fmt)
                p01 = e[0] * ra[0] + e[1] * ra[1]
                p23 = e[2] * rb[0] + e[3] * rb[1]
                scr[pl.ds(j * 16, 16)] = p01 + p23

            def col_sum():
                acc = plsc.load_gather(scr, [col0])
                for c in range(1, 16):
                    acc = acc + plsc.load_gather(scr, [col0 + c])
                return acc

            dot_to(0, lambda h: prows[i, pl.ds(h * 32, 32)])
            for k in range(15):
                dot_to(1 + k,
                       lambda h, k=k: nrows[i * _K + k, pl.ds(h * 32, 32)])
            outv[i, pl.ds(0, 16)] = col_sum()
            for k in range(15, _K):
                dot_to(k - 15,
                       lambda h, k=k: nrows[i * _K + k, pl.ds(h * 32, 32)])
            outv[i, pl.ds(16, 16)] = col_sum()
            return carry2

        lax.fori_loop(0, _T, sample_body, 0)
        pltpu.sync_copy(outv, out_hbm.at[pl.ds(tb, _T), :])

    issue(0, 0)

    def pair_body(g, carry):
        t0 = 2 * g
        issue(t0 + 1, 1)
        drain(t0, 0)
        compute(t0, 0)

        @pl.when(t0 + 2 < _NT)
        def _():
            issue(t0 + 2, 0)

        drain(t0 + 1, 1)
        compute(t0 + 1, 1)
        return carry

    lax.fori_loop(0, _NT // 2, pair_body, 0)


_sc_call = pl.kernel(
    _sc_logits_kernel,
    out_type=jax.ShapeDtypeStruct((_B, _OC), jnp.float32),
    mesh=plsc.VectorSubcoreMesh(core_axis_name="c", subcore_axis_name="s"),
    compiler_params=pltpu.CompilerParams(needs_layout_passes=False,
                                         use_tc_tiling_on_sc=False),
    scratch_types=[
        pltpu.VMEM((_BPW,), jnp.int32),
        pltpu.VMEM((_BPW,), jnp.int32),
        pltpu.VMEM((_BPW * _K,), jnp.int32),
        pltpu.VMEM((_T, _D), jnp.bfloat16),
        pltpu.VMEM((_T, _D), jnp.bfloat16),
        pltpu.VMEM((_T, _D), jnp.bfloat16),
        pltpu.VMEM((_T, _D), jnp.bfloat16),
        pltpu.VMEM((_TK, _D), jnp.bfloat16),
        pltpu.VMEM((_TK, _D), jnp.bfloat16),
        pltpu.VMEM((_T, _OC), jnp.float32),
        pltpu.VMEM((256,), jnp.float32),
        pltpu.SemaphoreType.DMA,
        pltpu.SemaphoreType.DMA,
    ],
)


def _tc_loss_kernel(x_ref, o_ref):
    x = x_ref[...]
    col = lax.broadcasted_iota(jnp.int32, x.shape, 1) % _OC
    z = jnp.where(col == 0, x, -x)
    ls = jnp.minimum(z, 0.0) - jnp.log1p(jnp.exp(-jnp.abs(z)))
    ls = jnp.where(col < _K + 1, ls, 0.0)
    o_ref[0, 0] = -jnp.sum(ls) / _B


@jax.jit
def kernel(src, pos, negs, embedder_W, context_W):
    logits = _sc_call(src.reshape(_B), pos.reshape(_B),
                      negs.reshape(_B * _K),
                      embedder_W.astype(jnp.bfloat16),
                      context_W.astype(jnp.bfloat16))
    x = logits.reshape(_B * _OC // 128, 128)
    loss = pl.pallas_call(
        _tc_loss_kernel,
        out_shape=jax.ShapeDtypeStruct((1, 1), jnp.float32),
        out_specs=pl.BlockSpec(memory_space=pltpu.SMEM),
    )(x)
    return loss[0, 0]


# speedup vs baseline: 1.7948x; 1.7948x over previous
import jax
import jax.numpy as jnp
from jax import lax
from jax.experimental import pallas as pl
from jax.experimental.pallas import tpu as pltpu
from jax.experimental.pallas import tpu_sc as plsc

_B = 16384
_K = 20
_D = 64
_NC = 2
_NS = 16
_NW = _NC * _NS
_BPW = _B // _NW
_T = 32
_NT = _BPW // _T
_TK = _T * _K
_NCHUNK = _TK // 128
_OC = 32


def _sc_logits_kernel(src_hbm, pos_hbm, negs_hbm, emb_hbm, ctx_hbm, out_hbm,
                      sidx, pidx, nidx,
                      erows0, erows1, prows0, prows1, nrows0, nrows1,
                      outv, scr, sem0, sem1):
    wid = lax.axis_index("s") * _NC + lax.axis_index("c")
    base = wid * _BPW
    col0 = lax.iota(jnp.int32, 16) * 16
    bufs = ((erows0, prows0, nrows0, sem0),
            (erows1, prows1, nrows1, sem1))

    pltpu.sync_copy(src_hbm.at[pl.ds(base, _BPW)], sidx)
    pltpu.sync_copy(pos_hbm.at[pl.ds(base, _BPW)], pidx)
    pltpu.sync_copy(negs_hbm.at[pl.ds(base * _K, _BPW * _K)], nidx)

    def gather_descs(t, b):
        erows, prows, nrows, sem = bufs[b]
        descs = [(emb_hbm.at[sidx.at[pl.ds(t * _T, _T)]], erows, sem),
                 (ctx_hbm.at[pidx.at[pl.ds(t * _T, _T)]], prows, sem)]
        for c in range(_NCHUNK):
            descs.append(
                (ctx_hbm.at[nidx.at[pl.ds(t * _TK + c * 128, 128)]],
                 nrows.at[pl.ds(c * 128, 128)], sem))
        return descs

    def issue(t, b):
        for d in gather_descs(t, b):
            pltpu.async_copy(*d)

    def drain(t, b):
        for d in gather_descs(t, b):
            pltpu.make_async_copy(*d).wait()

    def compute(t, b):
        erows, prows, nrows, _ = bufs[b]
        tb = base + t * _T

        def sample_body(i, carry2):
            e = [erows[i, pl.ds(c * 16, 16)] for c in range(4)]

            def dot_to(j, load_r):
                p01 = e[0] * load_r(0) + e[1] * load_r(1)
                p23 = e[2] * load_r(2) + e[3] * load_r(3)
                scr[pl.ds(j * 16, 16)] = p01 + p23

            def col_sum():
                acc = plsc.load_gather(scr, [col0])
                for c in range(1, 16):
                    acc = acc + plsc.load_gather(scr, [col0 + c])
                return acc

            dot_to(0, lambda c: prows[i, pl.ds(c * 16, 16)])
            for k in range(15):
                dot_to(1 + k,
                       lambda c, k=k: nrows[i * _K + k, pl.ds(c * 16, 16)])
            outv[i, pl.ds(0, 16)] = col_sum()
            for k in range(15, _K):
                dot_to(k - 15,
                       lambda c, k=k: nrows[i * _K + k, pl.ds(c * 16, 16)])
            outv[i, pl.ds(16, 16)] = col_sum()
            return carry2

        lax.fori_loop(0, _T, sample_body, 0)
        pltpu.sync_copy(outv, out_hbm.at[pl.ds(tb, _T), :])

    issue(0, 0)

    def pair_body(g, carry):
        t0 = 2 * g
        issue(t0 + 1, 1)
        drain(t0, 0)
        compute(t0, 0)

        @pl.when(t0 + 2 < _NT)
        def _():
            issue(t0 + 2, 0)

        drain(t0 + 1, 1)
        compute(t0 + 1, 1)
        return carry

    lax.fori_loop(0, _NT // 2, pair_body, 0)


_sc_call = pl.kernel(
    _sc_logits_kernel,
    out_type=jax.ShapeDtypeStruct((_B, _OC), jnp.float32),
    mesh=plsc.VectorSubcoreMesh(core_axis_name="c", subcore_axis_name="s"),
    compiler_params=pltpu.CompilerParams(needs_layout_passes=False,
                                         use_tc_tiling_on_sc=False),
    scratch_types=[
        pltpu.VMEM((_BPW,), jnp.int32),
        pltpu.VMEM((_BPW,), jnp.int32),
        pltpu.VMEM((_BPW * _K,), jnp.int32),
        pltpu.VMEM((_T, _D), jnp.float32),
        pltpu.VMEM((_T, _D), jnp.float32),
        pltpu.VMEM((_T, _D), jnp.float32),
        pltpu.VMEM((_T, _D), jnp.float32),
        pltpu.VMEM((_TK, _D), jnp.float32),
        pltpu.VMEM((_TK, _D), jnp.float32),
        pltpu.VMEM((_T, _OC), jnp.float32),
        pltpu.VMEM((256,), jnp.float32),
        pltpu.SemaphoreType.DMA,
        pltpu.SemaphoreType.DMA,
    ],
)


def _tc_loss_kernel(x_ref, o_ref):
    x = x_ref[...]
    col = lax.broadcasted_iota(jnp.int32, x.shape, 1) % _OC
    z = jnp.where(col == 0, x, -x)
    ls = jnp.minimum(z, 0.0) - jnp.log1p(jnp.exp(-jnp.abs(z)))
    ls = jnp.where(col < _K + 1, ls, 0.0)
    o_ref[0, 0] = -jnp.sum(ls) / _B


@jax.jit
def kernel(src, pos, negs, embedder_W, context_W):
    logits = _sc_call(src.reshape(_B), pos.reshape(_B),
                      negs.reshape(_B * _K),
                      embedder_W, context_W)
    x = logits.reshape(_B * _OC // 128, 128)
    loss = pl.pallas_call(
        _tc_loss_kernel,
        out_shape=jax.ShapeDtypeStruct((1, 1), jnp.float32),
        out_specs=pl.BlockSpec(memory_space=pltpu.SMEM),
    )(x)
    return loss[0, 0]
